# baseline (speedup 1.0000x reference)
import functools

import jax
import jax.numpy as jnp
from jax.experimental import pallas as pl


def _body(x_ref, adj_ref, w_ref, out_ref):
    h = jnp.dot(adj_ref[...], x_ref[...], preferred_element_type=jnp.float32)
    out_ref[...] = jax.lax.dot_general(
        h, w_ref[...], (((1,), (1,)), ((), ())),
        preferred_element_type=jnp.float32,
    )


@functools.partial(jax.jit, static_argnames=("m_blk", "interpret"))
def _graph_conv(x, adj, W, *, m_blk, interpret=False):
    n, d_in = x.shape
    d_out = W.shape[0]
    return pl.pallas_call(
        _body,
        grid=(n // m_blk,),
        in_specs=[
            pl.BlockSpec((n, d_in), lambda m: (0, 0)),
            pl.BlockSpec((m_blk, n), lambda m: (m, 0)),
            pl.BlockSpec((d_out, d_in), lambda m: (0, 0)),
        ],
        out_specs=pl.BlockSpec((m_blk, d_out), lambda m: (m, 0)),
        out_shape=jax.ShapeDtypeStruct((n, d_out), jnp.float32),
        interpret=interpret,
    )(x, adj, W)


def kernel(x, adj, W):
    n = x.shape[0]
    m_blk = 200 if n % 200 == 0 else n
    return _graph_conv(x, adj, W, m_blk=m_blk)


# speedup vs baseline: 1.0196x; 1.0196x over previous
import functools

import jax
import jax.numpy as jnp
from jax.experimental import pallas as pl


def _body(x_ref, adj_ref, w_ref, out_ref):
    h = jnp.dot(adj_ref[...], x_ref[...], preferred_element_type=jnp.float32)
    out_ref[...] = jax.lax.dot_general(
        h, w_ref[...], (((1,), (1,)), ((), ())),
        preferred_element_type=jnp.float32,
    )


@functools.partial(jax.jit, static_argnames=("m_blk", "interpret"))
def _graph_conv(x, adj, W, *, m_blk, interpret=False):
    n, d_in = x.shape
    d_out = W.shape[0]
    return pl.pallas_call(
        _body,
        grid=(n // m_blk,),
        in_specs=[
            pl.BlockSpec((n, d_in), lambda m: (0, 0)),
            pl.BlockSpec((m_blk, n), lambda m: (m, 0)),
            pl.BlockSpec((d_out, d_in), lambda m: (0, 0)),
        ],
        out_specs=pl.BlockSpec((m_blk, d_out), lambda m: (m, 0)),
        out_shape=jax.ShapeDtypeStruct((n, d_out), jnp.float32),
        interpret=interpret,
    )(x, adj, W)


def kernel(x, adj, W):
    n = x.shape[0]
    m_blk = 400 if n % 400 == 0 else n
    return _graph_conv(x, adj, W, m_blk=m_blk)
